# Initial kernel scaffold; baseline (speedup 1.0000x reference)
import jax
import jax.numpy as jnp
from jax.experimental import pallas as pl


def kernel(x, edge_index, W1, b1, W2, b2, W3, b3, Wc, bc):
    raise NotImplementedError("write your pallas kernel here")


# speedup vs baseline: 14.9444x; 14.9444x over previous
import functools

import jax
import jax.numpy as jnp
from jax import lax
from jax.experimental import pallas as pl
from jax.experimental.pallas import tpu as pltpu
from jax.experimental.pallas import tpu_sc as plsc

N_IN = 10000
E_IN = 320000
NP = 10240
F = 128
NC, NS, L = 2, 16, 16
NW = NC * NS
C = 128
NCHUNK = E_IN // C
BASE_CH = NCHUNK // NW
EXTRA = NCHUNK - BASE_CH * NW
RPT = NP // NS
EPT = E_IN // NW

_MESH = plsc.VectorSubcoreMesh(core_axis_name="c", subcore_axis_name="s")


def _make_hist():
    @functools.partial(
        pl.kernel,
        out_type=jax.ShapeDtypeStruct((NC * NP,), jnp.float32),
        mesh=_MESH,
        scratch_types=[
            pltpu.VMEM((C,), jnp.int32),
            pltpu.VMEM((C,), jnp.float32),
            pltpu.VMEM((RPT,), jnp.float32),
            pltpu.VMEM_SHARED((NP,), jnp.float32),
        ],
    )
    def hist(dst_hbm, deg_hbm, didx, ones, buf, deg_sh):
        cid = lax.axis_index("c")
        sid = lax.axis_index("s")
        wid = sid * NC + cid

        def zb(t, _):
            buf[pl.ds(t * L, L)] = jnp.zeros((L,), jnp.float32)
            return 0
        lax.fori_loop(0, RPT // L, zb, 0)

        def ob(t, _):
            ones[pl.ds(t * L, L)] = jnp.ones((L,), jnp.float32)
            return 0
        lax.fori_loop(0, C // L, ob, 0)

        rbase = sid * RPT
        pltpu.sync_copy(buf, deg_sh.at[pl.ds(rbase, RPT)])
        plsc.subcore_barrier()

        def do_chunk(ch):
            pltpu.sync_copy(dst_hbm.at[pl.ds(ch * C, C)], didx)
            pltpu.sync_copy(ones, deg_sh.at[didx], add=True)

        def body(j, _):
            do_chunk(wid * BASE_CH + j)
            return 0
        lax.fori_loop(0, BASE_CH, body, 0)

        @pl.when(wid < EXTRA)
        def _():
            do_chunk(NW * BASE_CH + wid)

        plsc.subcore_barrier()
        pltpu.sync_copy(deg_sh.at[pl.ds(rbase, RPT)], buf)
        pltpu.sync_copy(buf, deg_hbm.at[pl.ds(cid * NP + rbase, RPT)])

    return hist


def _make_segsum():
    grp = F // L

    @functools.partial(
        pl.kernel,
        out_type=jax.ShapeDtypeStruct((NC * NP, F), jnp.float32),
        mesh=_MESH,
        scratch_types=[
            pltpu.VMEM((C,), jnp.int32),
            pltpu.VMEM((C,), jnp.int32),
            pltpu.VMEM((C, F), jnp.float32),
            pltpu.VMEM_SHARED((NP, F), jnp.float32),
            pltpu.SemaphoreType.DMA,
        ],
    )
    def seg(src_hbm, dst_hbm, y_hbm, agg_hbm, sidx, didx, rows, agg_sh, sem):
        cid = lax.axis_index("c")
        sid = lax.axis_index("s")
        wid = sid * NC + cid

        def zstore(t, _):
            i = t // grp
            j = t % grp
            rows[i, pl.ds(j * L, L)] = jnp.zeros((L,), jnp.float32)
            return 0
        lax.fori_loop(0, C * grp, zstore, 0)

        rbase = sid * RPT

        def zcp(k, _):
            pltpu.sync_copy(rows, agg_sh.at[pl.ds(rbase + k * C, C)])
            return 0
        lax.fori_loop(0, RPT // C, zcp, 0)
        plsc.subcore_barrier()

        def do_chunk(ch):
            off = ch * C
            pltpu.sync_copy(src_hbm.at[pl.ds(off, C)], sidx)
            pltpu.sync_copy(dst_hbm.at[pl.ds(off, C)], didx)
            pltpu.async_copy(y_hbm.at[sidx], rows, sem).wait()
            pltpu.sync_copy(rows, agg_sh.at[didx], add=True)

        def body(j, _):
            do_chunk(wid * BASE_CH + j)
            return 0
        lax.fori_loop(0, BASE_CH, body, 0)

        @pl.when(wid < EXTRA)
        def _():
            do_chunk(NW * BASE_CH + wid)

        plsc.subcore_barrier()

        def ocp(k, _):
            r = rbase + k * C
            pltpu.sync_copy(agg_sh.at[pl.ds(r, C)], rows)
            pltpu.sync_copy(rows, agg_hbm.at[pl.ds(cid * NP + r, C)])
            return 0
        lax.fori_loop(0, RPT // C, ocp, 0)

    return seg


def _make_edge():
    @functools.partial(
        pl.kernel,
        out_type=jax.ShapeDtypeStruct((E_IN,), jnp.float32),
        mesh=_MESH,
        scratch_types=[
            pltpu.VMEM((EPT,), jnp.int32),
            pltpu.VMEM((EPT,), jnp.int32),
            pltpu.VMEM((EPT,), jnp.float32),
            pltpu.VMEM((2 * NP,), jnp.float32),
        ],
        compiler_params=pltpu.CompilerParams(needs_layout_passes=False),
    )
    def edge(src_hbm, dst_hbm, pq_hbm, out_hbm, sidx, didx, outv, pqv):
        cid = lax.axis_index("c")
        sid = lax.axis_index("s")
        wid = sid * NC + cid
        base = wid * EPT
        pltpu.sync_copy(pq_hbm, pqv)
        pltpu.sync_copy(src_hbm.at[pl.ds(base, EPT)], sidx)
        pltpu.sync_copy(dst_hbm.at[pl.ds(base, EPT)], didx)

        def body(i, _):
            s16 = sidx[pl.ds(i * L, L)]
            d16 = didx[pl.ds(i * L, L)]
            pv = plsc.load_gather(pqv, [s16 * 2])
            qv = plsc.load_gather(pqv, [d16 * 2 + 1])
            outv[pl.ds(i * L, L)] = pv + qv
            return 0
        lax.fori_loop(0, EPT // L, body, 0)
        pltpu.sync_copy(outv, out_hbm.at[pl.ds(base, EPT)])

    return edge


_R = 1024
_GRID = NP // _R


def _prep_tc(deg_parts, x_p, W1):
    def body(deg_ref, x_ref, w_ref, y_ref):
        d = deg_ref[0, :] + deg_ref[1, :] + 1.0
        dinv = lax.rsqrt(d)
        xw = jnp.dot(x_ref[...], w_ref[...], preferred_element_type=jnp.float32)
        y_ref[...] = xw * dinv[:, None]

    return pl.pallas_call(
        body,
        grid=(_GRID,),
        in_specs=[
            pl.BlockSpec((2, _R), lambda i: (0, i)),
            pl.BlockSpec((_R, F), lambda i: (i, 0)),
            pl.BlockSpec((F, F), lambda i: (0, 0)),
        ],
        out_specs=pl.BlockSpec((_R, F), lambda i: (i, 0)),
        out_shape=jax.ShapeDtypeStruct((NP, F), jnp.float32),
    )(deg_parts, x_p, W1)


def _mid_tc(deg_parts, agg, y, b, Wn):
    def body(deg_ref, agg_ref, y_ref, b_ref, w_ref, o_ref):
        d = deg_ref[0, :] + deg_ref[1, :] + 1.0
        dinv = lax.rsqrt(d)
        tot = agg_ref[0] + agg_ref[1] + y_ref[...]
        h = jnp.maximum(tot * dinv[:, None] + b_ref[0, :][None, :], 0.0)
        hw = jnp.dot(h, w_ref[...], preferred_element_type=jnp.float32)
        o_ref[...] = hw * dinv[:, None]

    return pl.pallas_call(
        body,
        grid=(_GRID,),
        in_specs=[
            pl.BlockSpec((2, _R), lambda i: (0, i)),
            pl.BlockSpec((NC, _R, F), lambda i: (0, i, 0)),
            pl.BlockSpec((_R, F), lambda i: (i, 0)),
            pl.BlockSpec((1, F), lambda i: (0, 0)),
            pl.BlockSpec((F, F), lambda i: (0, 0)),
        ],
        out_specs=pl.BlockSpec((_R, F), lambda i: (i, 0)),
        out_shape=jax.ShapeDtypeStruct((NP, F), jnp.float32),
    )(deg_parts, agg, y, b, Wn)


def _final_tc(deg_parts, agg, y, b, wc2, bc2):
    def body(deg_ref, agg_ref, y_ref, b_ref, w_ref, bc_ref, o_ref):
        d = deg_ref[0, :] + deg_ref[1, :] + 1.0
        dinv = lax.rsqrt(d)
        tot = agg_ref[0] + agg_ref[1] + y_ref[...]
        h = jnp.maximum(tot * dinv[:, None] + b_ref[0, :][None, :], 0.0)
        pq = jnp.dot(h, w_ref[...], preferred_element_type=jnp.float32)
        o_ref[...] = pq + bc_ref[0, :][None, :]

    return pl.pallas_call(
        body,
        grid=(_GRID,),
        in_specs=[
            pl.BlockSpec((2, _R), lambda i: (0, i)),
            pl.BlockSpec((NC, _R, F), lambda i: (0, i, 0)),
            pl.BlockSpec((_R, F), lambda i: (i, 0)),
            pl.BlockSpec((1, F), lambda i: (0, 0)),
            pl.BlockSpec((F, 2), lambda i: (0, 0)),
            pl.BlockSpec((1, 2), lambda i: (0, 0)),
        ],
        out_specs=pl.BlockSpec((_R, 2), lambda i: (i, 0)),
        out_shape=jax.ShapeDtypeStruct((NP, 2), jnp.float32),
    )(deg_parts, agg, y, b, wc2, bc2)


_hist_k = _make_hist()
_seg_k = _make_segsum()
_edge_k = _make_edge()


def _pad2(w, rows, cols):
    return jnp.pad(w, ((0, rows - w.shape[0]), (0, cols - w.shape[1])))


def kernel(x, edge_index, W1, b1, W2, b2, W3, b3, Wc, bc):
    src = edge_index[0].astype(jnp.int32)
    dst = edge_index[1].astype(jnp.int32)
    x_p = jnp.pad(x, ((0, NP - x.shape[0]), (0, 0)))
    W2p = _pad2(W2, F, F)
    W3p = _pad2(W3, F, F)
    b2p = jnp.pad(b2, (0, F - b2.shape[0])).reshape(1, F)
    b3p = jnp.pad(b3, (0, F - b3.shape[0])).reshape(1, F)
    wc2 = _pad2(jnp.concatenate([Wc[:32], Wc[32:]], axis=1), F, 2)
    bc2 = jnp.concatenate([bc, jnp.zeros((1,), jnp.float32)]).reshape(1, 2)

    deg_parts = _hist_k(dst).reshape(NC, NP)
    y1 = _prep_tc(deg_parts, x_p, W1)
    agg1 = _seg_k(src, dst, y1).reshape(NC, NP, F)
    y2 = _mid_tc(deg_parts, agg1, y1, b1.reshape(1, F), W2p)
    agg2 = _seg_k(src, dst, y2).reshape(NC, NP, F)
    y3 = _mid_tc(deg_parts, agg2, y2, b2p, W3p)
    agg3 = _seg_k(src, dst, y3).reshape(NC, NP, F)
    pq = _final_tc(deg_parts, agg3, y3, b3p, wc2, bc2)
    scores = _edge_k(src, dst, pq.reshape(2 * NP))
    return scores.reshape(E_IN, 1)


# speedup vs baseline: 25.9128x; 1.7339x over previous
import functools

import jax
import jax.numpy as jnp
from jax import lax
from jax.experimental import pallas as pl
from jax.experimental.pallas import tpu as pltpu
from jax.experimental.pallas import tpu_sc as plsc

N_IN = 10000
E_IN = 320000
NP = 10240
F = 128
NC, NS, L = 2, 16, 16
NW = NC * NS
C = 64
CPT = 160
HCPT = CPT // 4
PH = CPT // HCPT
EPAD = C * CPT * NW
NB = 2
NIT = HCPT // (2 * NB)
RPT = NP // NS
ZC = 64
EPT = E_IN // NW

_MESH = plsc.VectorSubcoreMesh(core_axis_name="c", subcore_axis_name="s")


def _make_hist():
    @functools.partial(
        pl.kernel,
        out_type=jax.ShapeDtypeStruct((NC * NP,), jnp.float32),
        mesh=_MESH,
        scratch_types=[
            pltpu.VMEM((CPT, C), jnp.int32),
            pltpu.VMEM((C,), jnp.float32),
            pltpu.VMEM((RPT,), jnp.float32),
            pltpu.VMEM_SHARED((NP,), jnp.float32),
        ] + [pltpu.SemaphoreType.DMA] * 8,
    )
    def hist(dst_hbm, deg_hbm, didx, ones, buf, deg_sh, *sems):
        cid = lax.axis_index("c")
        sid = lax.axis_index("s")
        wid = sid * NC + cid

        def zb(t, _):
            buf[pl.ds(t * L, L)] = jnp.zeros((L,), jnp.float32)
            return 0
        lax.fori_loop(0, RPT // L, zb, 0)

        def ob(t, _):
            ones[pl.ds(t * L, L)] = jnp.ones((L,), jnp.float32)
            return 0
        lax.fori_loop(0, C // L, ob, 0)

        pltpu.sync_copy(dst_hbm.at[pl.ds(wid * CPT, CPT)], didx)
        rbase = sid * RPT
        pltpu.sync_copy(buf, deg_sh.at[pl.ds(rbase, RPT)])
        plsc.subcore_barrier()

        def body(g, _):
            for u in range(8):
                pltpu.async_copy(ones, deg_sh.at[didx.at[g * 8 + u]], sems[u], add=True)
            for u in range(8):
                pltpu.make_async_copy(ones, deg_sh.at[didx.at[g * 8 + u]], sems[u]).wait()
            return 0
        lax.fori_loop(0, CPT // 8, body, 0)

        plsc.subcore_barrier()
        pltpu.sync_copy(deg_sh.at[pl.ds(rbase, RPT)], buf)
        pltpu.sync_copy(buf, deg_hbm.at[pl.ds(cid * NP + rbase, RPT)])

    return hist


def _make_segsum():
    grp = F // L

    @functools.partial(
        pl.kernel,
        out_type=jax.ShapeDtypeStruct((NC * NP, F), jnp.float32),
        mesh=_MESH,
        scratch_types=[
            pltpu.VMEM((HCPT, C), jnp.int32),
            pltpu.VMEM((HCPT, C), jnp.int32),
            pltpu.VMEM_SHARED((NP, F), jnp.float32),
        ] + [pltpu.VMEM((C, F), jnp.float32)] * (2 * NB)
          + [pltpu.SemaphoreType.DMA] * (4 * NB),
    )
    def seg(src_hbm, dst_hbm, y_hbm, agg_hbm, sidx, didx, agg_sh, *bufs_sems):
        rows = bufs_sems[: 2 * NB]
        semg = bufs_sems[2 * NB: 3 * NB]
        semgb = bufs_sems[3 * NB: 4 * NB]
        sems = bufs_sems[4 * NB: 5 * NB]
        semsb = bufs_sems[5 * NB: 6 * NB]
        cid = lax.axis_index("c")
        sid = lax.axis_index("s")
        wid = sid * NC + cid

        def zstore(t, _):
            i = t // grp
            j = t % grp
            rows[0][i, pl.ds(j * L, L)] = jnp.zeros((L,), jnp.float32)
            return 0
        lax.fori_loop(0, C * grp, zstore, 0)

        rbase = sid * RPT

        def zcp(k, _):
            pltpu.sync_copy(rows[0], agg_sh.at[pl.ds(rbase + k * ZC, ZC)])
            return 0
        lax.fori_loop(0, RPT // ZC, zcp, 0)
        plsc.subcore_barrier()

        A = rows[:NB]
        B = rows[NB:]

        def g_start(buf, sem, t):
            pltpu.async_copy(y_hbm.at[sidx.at[t]], buf, sem)

        def g_wait(buf, sem, t):
            pltpu.make_async_copy(y_hbm.at[sidx.at[t]], buf, sem).wait()

        def s_start(buf, sem, t):
            pltpu.async_copy(buf, agg_sh.at[didx.at[t]], sem, add=True)

        def s_wait(buf, sem, t):
            pltpu.make_async_copy(buf, agg_sh.at[didx.at[t]], sem).wait()

        for ph in range(PH):
            pltpu.sync_copy(src_hbm.at[pl.ds(wid * CPT + ph * HCPT, HCPT)], sidx)
            pltpu.sync_copy(dst_hbm.at[pl.ds(wid * CPT + ph * HCPT, HCPT)], didx)
            for b in range(NB):
                g_start(A[b], semg[b], b)

            def body(i, _):
                base = 2 * NB * i
                for b in range(NB):
                    g_wait(A[b], semg[b], base + b)
                    s_start(A[b], sems[b], base + b)

                @pl.when(i > 0)
                def _():
                    for b in range(NB):
                        s_wait(B[b], semsb[b], base - NB + b)
                for b in range(NB):
                    g_start(B[b], semgb[b], base + NB + b)

                for b in range(NB):
                    g_wait(B[b], semgb[b], base + NB + b)
                    s_start(B[b], semsb[b], base + NB + b)

                for b in range(NB):
                    s_wait(A[b], sems[b], base + b)

                @pl.when(i < NIT - 1)
                def _():
                    for b in range(NB):
                        g_start(A[b], semg[b], base + 2 * NB + b)
                return 0
            lax.fori_loop(0, NIT, body, 0)
            for b in range(NB):
                s_wait(B[b], semsb[b], 2 * NB * (NIT - 1) + NB + b)

        plsc.subcore_barrier()

        def ocp(k, _):
            r = rbase + k * ZC
            pltpu.sync_copy(agg_sh.at[pl.ds(r, ZC)], rows[0])
            pltpu.sync_copy(rows[0], agg_hbm.at[pl.ds(cid * NP + r, ZC)])
            return 0
        lax.fori_loop(0, RPT // ZC, ocp, 0)

    return seg


def _make_edge():
    @functools.partial(
        pl.kernel,
        out_type=jax.ShapeDtypeStruct((E_IN,), jnp.float32),
        mesh=_MESH,
        scratch_types=[
            pltpu.VMEM((EPT,), jnp.int32),
            pltpu.VMEM((EPT,), jnp.int32),
            pltpu.VMEM((EPT,), jnp.float32),
            pltpu.VMEM((2 * NP,), jnp.float32),
        ],
        compiler_params=pltpu.CompilerParams(needs_layout_passes=False),
    )
    def edge(src_hbm, dst_hbm, pq_hbm, out_hbm, sidx, didx, outv, pqv):
        cid = lax.axis_index("c")
        sid = lax.axis_index("s")
        wid = sid * NC + cid
        base = wid * EPT
        pltpu.sync_copy(pq_hbm, pqv)
        pltpu.sync_copy(src_hbm.at[pl.ds(base, EPT)], sidx)
        pltpu.sync_copy(dst_hbm.at[pl.ds(base, EPT)], didx)

        def body(i, _):
            s16 = sidx[pl.ds(i * L, L)]
            d16 = didx[pl.ds(i * L, L)]
            pv = plsc.load_gather(pqv, [s16 * 2])
            qv = plsc.load_gather(pqv, [d16 * 2 + 1])
            outv[pl.ds(i * L, L)] = pv + qv
            return 0
        lax.fori_loop(0, EPT // L, body, 0)
        pltpu.sync_copy(outv, out_hbm.at[pl.ds(base, EPT)])

    return edge


_R = 1024
_GRID = NP // _R


def _prep_tc(deg_parts, x_p, W1):
    def body(deg_ref, x_ref, w_ref, y_ref):
        d = deg_ref[0, :] + deg_ref[1, :] + 1.0
        dinv = lax.rsqrt(d)
        xw = jnp.dot(x_ref[...], w_ref[...], preferred_element_type=jnp.float32)
        y_ref[...] = xw * dinv[:, None]

    return pl.pallas_call(
        body,
        grid=(_GRID,),
        in_specs=[
            pl.BlockSpec((2, _R), lambda i: (0, i)),
            pl.BlockSpec((_R, F), lambda i: (i, 0)),
            pl.BlockSpec((F, F), lambda i: (0, 0)),
        ],
        out_specs=pl.BlockSpec((_R, F), lambda i: (i, 0)),
        out_shape=jax.ShapeDtypeStruct((NP, F), jnp.float32),
    )(deg_parts, x_p, W1)


def _mid_tc(deg_parts, agg, y, b, Wn):
    def body(deg_ref, agg_ref, y_ref, b_ref, w_ref, o_ref):
        d = deg_ref[0, :] + deg_ref[1, :] + 1.0
        dinv = lax.rsqrt(d)
        tot = agg_ref[0] + agg_ref[1] + y_ref[...]
        h = jnp.maximum(tot * dinv[:, None] + b_ref[0, :][None, :], 0.0)
        hw = jnp.dot(h, w_ref[...], preferred_element_type=jnp.float32)
        o_ref[...] = hw * dinv[:, None]

    return pl.pallas_call(
        body,
        grid=(_GRID,),
        in_specs=[
            pl.BlockSpec((2, _R), lambda i: (0, i)),
            pl.BlockSpec((NC, _R, F), lambda i: (0, i, 0)),
            pl.BlockSpec((_R, F), lambda i: (i, 0)),
            pl.BlockSpec((1, F), lambda i: (0, 0)),
            pl.BlockSpec((F, F), lambda i: (0, 0)),
        ],
        out_specs=pl.BlockSpec((_R, F), lambda i: (i, 0)),
        out_shape=jax.ShapeDtypeStruct((NP, F), jnp.float32),
    )(deg_parts, agg, y, b, Wn)


def _final_tc(deg_parts, agg, y, b, wc2, bc2):
    def body(deg_ref, agg_ref, y_ref, b_ref, w_ref, bc_ref, o_ref):
        d = deg_ref[0, :] + deg_ref[1, :] + 1.0
        dinv = lax.rsqrt(d)
        tot = agg_ref[0] + agg_ref[1] + y_ref[...]
        h = jnp.maximum(tot * dinv[:, None] + b_ref[0, :][None, :], 0.0)
        pq = jnp.dot(h, w_ref[...], preferred_element_type=jnp.float32)
        o_ref[...] = pq + bc_ref[0, :][None, :]

    return pl.pallas_call(
        body,
        grid=(_GRID,),
        in_specs=[
            pl.BlockSpec((2, _R), lambda i: (0, i)),
            pl.BlockSpec((NC, _R, F), lambda i: (0, i, 0)),
            pl.BlockSpec((_R, F), lambda i: (i, 0)),
            pl.BlockSpec((1, F), lambda i: (0, 0)),
            pl.BlockSpec((F, 2), lambda i: (0, 0)),
            pl.BlockSpec((1, 2), lambda i: (0, 0)),
        ],
        out_specs=pl.BlockSpec((_R, 2), lambda i: (i, 0)),
        out_shape=jax.ShapeDtypeStruct((NP, 2), jnp.float32),
    )(deg_parts, agg, y, b, wc2, bc2)


_hist_k = _make_hist()
_seg_k = _make_segsum()
_edge_k = _make_edge()


def _pad2(w, rows, cols):
    return jnp.pad(w, ((0, rows - w.shape[0]), (0, cols - w.shape[1])))


def kernel(x, edge_index, W1, b1, W2, b2, W3, b3, Wc, bc):
    src = edge_index[0].astype(jnp.int32)
    dst = edge_index[1].astype(jnp.int32)
    pad_idx = (N_IN + jnp.arange(EPAD - E_IN, dtype=jnp.int32) % (NP - N_IN))
    src_p = jnp.concatenate([src, pad_idx]).reshape(NW * CPT, C)
    dst_p = jnp.concatenate([dst, pad_idx]).reshape(NW * CPT, C)
    x_p = jnp.pad(x, ((0, NP - x.shape[0]), (0, 0)))
    W2p = _pad2(W2, F, F)
    W3p = _pad2(W3, F, F)
    b2p = jnp.pad(b2, (0, F - b2.shape[0])).reshape(1, F)
    b3p = jnp.pad(b3, (0, F - b3.shape[0])).reshape(1, F)
    wc2 = _pad2(jnp.concatenate([Wc[:32], Wc[32:]], axis=1), F, 2)
    bc2 = jnp.concatenate([bc, jnp.zeros((1,), jnp.float32)]).reshape(1, 2)

    deg_parts = _hist_k(dst_p).reshape(NC, NP)
    y1 = _prep_tc(deg_parts, x_p, W1)
    agg1 = _seg_k(src_p, dst_p, y1).reshape(NC, NP, F)
    y2 = _mid_tc(deg_parts, agg1, y1, b1.reshape(1, F), W2p)
    agg2 = _seg_k(src_p, dst_p, y2).reshape(NC, NP, F)
    y3 = _mid_tc(deg_parts, agg2, y2, b2p, W3p)
    agg3 = _seg_k(src_p, dst_p, y3).reshape(NC, NP, F)
    pq = _final_tc(deg_parts, agg3, y3, b3p, wc2, bc2)
    scores = _edge_k(src, dst, pq.reshape(2 * NP))
    return scores.reshape(E_IN, 1)


# speedup vs baseline: 30.0513x; 1.1597x over previous
import functools

import jax
import jax.numpy as jnp
from jax import lax
from jax.experimental import pallas as pl
from jax.experimental.pallas import tpu as pltpu
from jax.experimental.pallas import tpu_sc as plsc

N_IN = 10000
E_IN = 320000
NP = 10240
F = 128
NC, NS, L = 2, 16, 16
NW = NC * NS
C = 64
CPT = 160
HCPT = CPT // 4
PH = CPT // HCPT
EPAD = C * CPT * NW
NB = 2
NIT = HCPT // (2 * NB)
RPT = NP // NS
ZC = 64
EPT = E_IN // NW

_MESH = plsc.VectorSubcoreMesh(core_axis_name="c", subcore_axis_name="s")


def _make_hist():
    @functools.partial(
        pl.kernel,
        out_type=jax.ShapeDtypeStruct((NC * NP,), jnp.float32),
        mesh=_MESH,
        scratch_types=[
            pltpu.VMEM((CPT, C), jnp.int32),
            pltpu.VMEM((C,), jnp.float32),
            pltpu.VMEM((RPT,), jnp.float32),
            pltpu.VMEM_SHARED((NP,), jnp.float32),
        ] + [pltpu.SemaphoreType.DMA] * 8,
    )
    def hist(dst_hbm, deg_hbm, didx, ones, buf, deg_sh, *sems):
        cid = lax.axis_index("c")
        sid = lax.axis_index("s")
        wid = sid * NC + cid

        def zb(t, _):
            buf[pl.ds(t * L, L)] = jnp.zeros((L,), jnp.float32)
            return 0
        lax.fori_loop(0, RPT // L, zb, 0)

        def ob(t, _):
            ones[pl.ds(t * L, L)] = jnp.ones((L,), jnp.float32)
            return 0
        lax.fori_loop(0, C // L, ob, 0)

        pltpu.sync_copy(dst_hbm.at[pl.ds(wid * CPT, CPT)], didx)
        rbase = sid * RPT
        pltpu.sync_copy(buf, deg_sh.at[pl.ds(rbase, RPT)])
        plsc.subcore_barrier()

        def body(g, _):
            for u in range(8):
                pltpu.async_copy(ones, deg_sh.at[didx.at[g * 8 + u]], sems[u], add=True)
            for u in range(8):
                pltpu.make_async_copy(ones, deg_sh.at[didx.at[g * 8 + u]], sems[u]).wait()
            return 0
        lax.fori_loop(0, CPT // 8, body, 0)

        plsc.subcore_barrier()
        pltpu.sync_copy(deg_sh.at[pl.ds(rbase, RPT)], buf)
        pltpu.sync_copy(buf, deg_hbm.at[pl.ds(cid * NP + rbase, RPT)])

    return hist


C2 = 128
CPT2 = EPAD // (C2 * NW)
HCPT2 = CPT2 // 4
NIT2 = HCPT2 // (2 * NB)
OC2 = 32


def _make_segsum_narrow(fr):
    grp = F // L
    ngrp = fr // L

    @functools.partial(
        pl.kernel,
        out_type=[
            jax.ShapeDtypeStruct((NC * NP, F), jnp.float32),
            jax.ShapeDtypeStruct((NP, fr), jnp.float32),
        ],
        mesh=_MESH,
        scratch_types=[
            pltpu.VMEM((HCPT2, C2), jnp.int32),
            pltpu.VMEM((HCPT2, C2), jnp.int32),
            pltpu.VMEM((OC2, F), jnp.float32),
            pltpu.VMEM_SHARED((NP, fr), jnp.float32),
        ] + [pltpu.VMEM((C2, fr), jnp.float32)] * (2 * NB)
          + [pltpu.SemaphoreType.DMA] * (4 * NB),
        compiler_params=pltpu.CompilerParams(use_tc_tiling_on_sc=False),
    )
    def seg(src_hbm, dst_hbm, y_hbm, agg_hbm, yn_hbm, sidx, didx, obuf,
            agg_sh, *rest):
        rows = rest[: 2 * NB]
        semg = rest[2 * NB: 3 * NB]
        semgb = rest[3 * NB: 4 * NB]
        sems = rest[4 * NB: 5 * NB]
        semsb = rest[5 * NB: 6 * NB]
        cid = lax.axis_index("c")
        sid = lax.axis_index("s")
        wid = sid * NC + cid
        rbase = sid * RPT

        def stage(k, _):
            r = rbase + k * OC2
            pltpu.sync_copy(y_hbm.at[pl.ds(r, OC2)], obuf)

            def rep(t, _):
                i = t // ngrp
                j = t % ngrp
                rows[0][i, pl.ds(j * L, L)] = obuf[i, pl.ds(j * L, L)]
                return 0
            lax.fori_loop(0, OC2 * ngrp, rep, 0)
            pltpu.sync_copy(rows[0].at[pl.ds(0, OC2)], yn_hbm.at[pl.ds(r, OC2)])
            return 0
        lax.fori_loop(0, RPT // OC2, stage, 0)

        def zstore(t, _):
            i = t // grp
            j = t % grp
            obuf[i, pl.ds(j * L, L)] = jnp.zeros((L,), jnp.float32)
            return 0
        lax.fori_loop(0, OC2 * grp, zstore, 0)

        def zrow(t, _):
            i = t // ngrp
            j = t % ngrp
            rows[1][i, pl.ds(j * L, L)] = jnp.zeros((L,), jnp.float32)
            return 0
        lax.fori_loop(0, C2 * ngrp, zrow, 0)

        def zcp(k, _):
            pltpu.sync_copy(rows[1], agg_sh.at[pl.ds(rbase + k * C2, C2)])
            return 0
        lax.fori_loop(0, RPT // C2, zcp, 0)
        plsc.subcore_barrier()

        A = rows[:NB]
        B = rows[NB:]

        def g_start(buf, sem, t):
            pltpu.async_copy(yn_hbm.at[sidx.at[t]], buf, sem)

        def g_wait(buf, sem, t):
            pltpu.make_async_copy(yn_hbm.at[sidx.at[t]], buf, sem).wait()

        def s_start(buf, sem, t):
            pltpu.async_copy(buf, agg_sh.at[didx.at[t]], sem, add=True)

        def s_wait(buf, sem, t):
            pltpu.make_async_copy(buf, agg_sh.at[didx.at[t]], sem).wait()

        for ph in range(PH):
            pltpu.sync_copy(src_hbm.at[pl.ds(wid * CPT2 + ph * HCPT2, HCPT2)], sidx)
            pltpu.sync_copy(dst_hbm.at[pl.ds(wid * CPT2 + ph * HCPT2, HCPT2)], didx)
            for b in range(NB):
                g_start(A[b], semg[b], b)

            def body(i, _):
                base = 2 * NB * i
                for b in range(NB):
                    g_wait(A[b], semg[b], base + b)
                    s_start(A[b], sems[b], base + b)

                @pl.when(i > 0)
                def _():
                    for b in range(NB):
                        s_wait(B[b], semsb[b], base - NB + b)
                for b in range(NB):
                    g_start(B[b], semgb[b], base + NB + b)

                for b in range(NB):
                    g_wait(B[b], semgb[b], base + NB + b)
                    s_start(B[b], semsb[b], base + NB + b)

                for b in range(NB):
                    s_wait(A[b], sems[b], base + b)

                @pl.when(i < NIT2 - 1)
                def _():
                    for b in range(NB):
                        g_start(A[b], semg[b], base + 2 * NB + b)
                return 0
            lax.fori_loop(0, NIT2, body, 0)
            for b in range(NB):
                s_wait(B[b], semsb[b], 2 * NB * (NIT2 - 1) + NB + b)

        plsc.subcore_barrier()

        def ocp(k, _):
            r = rbase + k * OC2
            pltpu.sync_copy(agg_sh.at[pl.ds(r, OC2)], rows[0].at[pl.ds(0, OC2)])

            def rep(t, _):
                i = t // ngrp
                j = t % ngrp
                obuf[i, pl.ds(j * L, L)] = rows[0][i, pl.ds(j * L, L)]
                return 0
            lax.fori_loop(0, OC2 * ngrp, rep, 0)
            pltpu.sync_copy(obuf, agg_hbm.at[pl.ds(cid * NP + r, OC2)])
            return 0
        lax.fori_loop(0, RPT // OC2, ocp, 0)

    return seg


def _make_segsum_full():
    grp = F // L

    @functools.partial(
        pl.kernel,
        out_type=jax.ShapeDtypeStruct((NC * NP, F), jnp.float32),
        mesh=_MESH,
        scratch_types=[
            pltpu.VMEM((HCPT, C), jnp.int32),
            pltpu.VMEM((HCPT, C), jnp.int32),
            pltpu.VMEM_SHARED((NP, F), jnp.float32),
        ] + [pltpu.VMEM((C, F), jnp.float32)] * (2 * NB)
          + [pltpu.SemaphoreType.DMA] * (4 * NB),
    )
    def seg(src_hbm, dst_hbm, y_hbm, agg_hbm, sidx, didx, agg_sh, *bufs_sems):
        rows = bufs_sems[: 2 * NB]
        semg = bufs_sems[2 * NB: 3 * NB]
        semgb = bufs_sems[3 * NB: 4 * NB]
        sems = bufs_sems[4 * NB: 5 * NB]
        semsb = bufs_sems[5 * NB: 6 * NB]
        cid = lax.axis_index("c")
        sid = lax.axis_index("s")
        wid = sid * NC + cid

        def zstore(t, _):
            i = t // grp
            j = t % grp
            rows[0][i, pl.ds(j * L, L)] = jnp.zeros((L,), jnp.float32)
            return 0
        lax.fori_loop(0, C * grp, zstore, 0)

        rbase = sid * RPT

        def zcp(k, _):
            pltpu.sync_copy(rows[0], agg_sh.at[pl.ds(rbase + k * ZC, ZC)])
            return 0
        lax.fori_loop(0, RPT // ZC, zcp, 0)
        plsc.subcore_barrier()

        A = rows[:NB]
        B = rows[NB:]

        def g_start(buf, sem, t):
            pltpu.async_copy(y_hbm.at[sidx.at[t]], buf, sem)

        def g_wait(buf, sem, t):
            pltpu.make_async_copy(y_hbm.at[sidx.at[t]], buf, sem).wait()

        def s_start(buf, sem, t):
            pltpu.async_copy(buf, agg_sh.at[didx.at[t]], sem, add=True)

        def s_wait(buf, sem, t):
            pltpu.make_async_copy(buf, agg_sh.at[didx.at[t]], sem).wait()

        for ph in range(PH):
            pltpu.sync_copy(src_hbm.at[pl.ds(wid * CPT + ph * HCPT, HCPT)], sidx)
            pltpu.sync_copy(dst_hbm.at[pl.ds(wid * CPT + ph * HCPT, HCPT)], didx)
            for b in range(NB):
                g_start(A[b], semg[b], b)

            def body(i, _):
                base = 2 * NB * i
                for b in range(NB):
                    g_wait(A[b], semg[b], base + b)
                    s_start(A[b], sems[b], base + b)

                @pl.when(i > 0)
                def _():
                    for b in range(NB):
                        s_wait(B[b], semsb[b], base - NB + b)
                for b in range(NB):
                    g_start(B[b], semgb[b], base + NB + b)

                for b in range(NB):
                    g_wait(B[b], semgb[b], base + NB + b)
                    s_start(B[b], semsb[b], base + NB + b)

                for b in range(NB):
                    s_wait(A[b], sems[b], base + b)

                @pl.when(i < NIT - 1)
                def _():
                    for b in range(NB):
                        g_start(A[b], semg[b], base + 2 * NB + b)
                return 0
            lax.fori_loop(0, NIT, body, 0)
            for b in range(NB):
                s_wait(B[b], semsb[b], 2 * NB * (NIT - 1) + NB + b)

        plsc.subcore_barrier()

        def ocp(k, _):
            r = rbase + k * ZC
            pltpu.sync_copy(agg_sh.at[pl.ds(r, ZC)], rows[0])
            pltpu.sync_copy(rows[0], agg_hbm.at[pl.ds(cid * NP + r, ZC)])
            return 0
        lax.fori_loop(0, RPT // ZC, ocp, 0)

    return seg


def _make_edge():
    @functools.partial(
        pl.kernel,
        out_type=jax.ShapeDtypeStruct((E_IN,), jnp.float32),
        mesh=_MESH,
        scratch_types=[
            pltpu.VMEM((EPT,), jnp.int32),
            pltpu.VMEM((EPT,), jnp.int32),
            pltpu.VMEM((EPT,), jnp.float32),
            pltpu.VMEM((2 * NP,), jnp.float32),
        ],
        compiler_params=pltpu.CompilerParams(needs_layout_passes=False),
    )
    def edge(src_hbm, dst_hbm, pq_hbm, out_hbm, sidx, didx, outv, pqv):
        cid = lax.axis_index("c")
        sid = lax.axis_index("s")
        wid = sid * NC + cid
        base = wid * EPT
        pltpu.sync_copy(pq_hbm, pqv)
        pltpu.sync_copy(src_hbm.at[pl.ds(base, EPT)], sidx)
        pltpu.sync_copy(dst_hbm.at[pl.ds(base, EPT)], didx)

        def body(i, _):
            s16 = sidx[pl.ds(i * L, L)]
            d16 = didx[pl.ds(i * L, L)]
            pv = plsc.load_gather(pqv, [s16 * 2])
            qv = plsc.load_gather(pqv, [d16 * 2 + 1])
            outv[pl.ds(i * L, L)] = pv + qv
            return 0
        lax.fori_loop(0, EPT // L, body, 0)
        pltpu.sync_copy(outv, out_hbm.at[pl.ds(base, EPT)])

    return edge


_R = 1024
_GRID = NP // _R


def _prep_tc(deg_parts, x_p, W1):
    def body(deg_ref, x_ref, w_ref, y_ref):
        d = deg_ref[0, :] + deg_ref[1, :] + 1.0
        dinv = lax.rsqrt(d)
        xw = jnp.dot(x_ref[...], w_ref[...], preferred_element_type=jnp.float32)
        y_ref[...] = xw * dinv[:, None]

    return pl.pallas_call(
        body,
        grid=(_GRID,),
        in_specs=[
            pl.BlockSpec((2, _R), lambda i: (0, i)),
            pl.BlockSpec((_R, F), lambda i: (i, 0)),
            pl.BlockSpec((F, F), lambda i: (0, 0)),
        ],
        out_specs=pl.BlockSpec((_R, F), lambda i: (i, 0)),
        out_shape=jax.ShapeDtypeStruct((NP, F), jnp.float32),
    )(deg_parts, x_p, W1)


def _mid_tc(deg_parts, agg, y, b, Wn):
    def body(deg_ref, agg_ref, y_ref, b_ref, w_ref, o_ref):
        d = deg_ref[0, :] + deg_ref[1, :] + 1.0
        dinv = lax.rsqrt(d)
        tot = agg_ref[0] + agg_ref[1] + y_ref[...]
        h = jnp.maximum(tot * dinv[:, None] + b_ref[0, :][None, :], 0.0)
        hw = jnp.dot(h, w_ref[...], preferred_element_type=jnp.float32)
        o_ref[...] = hw * dinv[:, None]

    return pl.pallas_call(
        body,
        grid=(_GRID,),
        in_specs=[
            pl.BlockSpec((2, _R), lambda i: (0, i)),
            pl.BlockSpec((NC, _R, F), lambda i: (0, i, 0)),
            pl.BlockSpec((_R, F), lambda i: (i, 0)),
            pl.BlockSpec((1, F), lambda i: (0, 0)),
            pl.BlockSpec((F, F), lambda i: (0, 0)),
        ],
        out_specs=pl.BlockSpec((_R, F), lambda i: (i, 0)),
        out_shape=jax.ShapeDtypeStruct((NP, F), jnp.float32),
    )(deg_parts, agg, y, b, Wn)


def _final_tc(deg_parts, agg, y, b, wc2, bc2):
    def body(deg_ref, agg_ref, y_ref, b_ref, w_ref, bc_ref, o_ref):
        d = deg_ref[0, :] + deg_ref[1, :] + 1.0
        dinv = lax.rsqrt(d)
        tot = agg_ref[0] + agg_ref[1] + y_ref[...]
        h = jnp.maximum(tot * dinv[:, None] + b_ref[0, :][None, :], 0.0)
        pq = jnp.dot(h, w_ref[...], preferred_element_type=jnp.float32)
        o_ref[...] = pq + bc_ref[0, :][None, :]

    return pl.pallas_call(
        body,
        grid=(_GRID,),
        in_specs=[
            pl.BlockSpec((2, _R), lambda i: (0, i)),
            pl.BlockSpec((NC, _R, F), lambda i: (0, i, 0)),
            pl.BlockSpec((_R, F), lambda i: (i, 0)),
            pl.BlockSpec((1, F), lambda i: (0, 0)),
            pl.BlockSpec((F, 2), lambda i: (0, 0)),
            pl.BlockSpec((1, 2), lambda i: (0, 0)),
        ],
        out_specs=pl.BlockSpec((_R, 2), lambda i: (i, 0)),
        out_shape=jax.ShapeDtypeStruct((NP, 2), jnp.float32),
    )(deg_parts, agg, y, b, wc2, bc2)


_hist_k = _make_hist()
_seg128 = _make_segsum_full()
_seg64 = _make_segsum_narrow(64)
_seg32 = _make_segsum_narrow(32)
_edge_k = _make_edge()


def _pad2(w, rows, cols):
    return jnp.pad(w, ((0, rows - w.shape[0]), (0, cols - w.shape[1])))


def kernel(x, edge_index, W1, b1, W2, b2, W3, b3, Wc, bc):
    src = edge_index[0].astype(jnp.int32)
    dst = edge_index[1].astype(jnp.int32)
    pad_idx = (N_IN + jnp.arange(EPAD - E_IN, dtype=jnp.int32) % (NP - N_IN))
    src_p = jnp.concatenate([src, pad_idx]).reshape(NW * CPT, C)
    dst_p = jnp.concatenate([dst, pad_idx]).reshape(NW * CPT, C)
    x_p = jnp.pad(x, ((0, NP - x.shape[0]), (0, 0)))
    W2p = _pad2(W2, F, F)
    W3p = _pad2(W3, F, F)
    b2p = jnp.pad(b2, (0, F - b2.shape[0])).reshape(1, F)
    b3p = jnp.pad(b3, (0, F - b3.shape[0])).reshape(1, F)
    wc2 = _pad2(jnp.concatenate([Wc[:32], Wc[32:]], axis=1), F, 2)
    bc2 = jnp.concatenate([bc, jnp.zeros((1,), jnp.float32)]).reshape(1, 2)

    deg_parts = _hist_k(dst_p).reshape(NC, NP)
    y1 = _prep_tc(deg_parts, x_p, W1)
    src_p2 = src_p.reshape(NW * CPT2, C2)
    dst_p2 = dst_p.reshape(NW * CPT2, C2)
    agg1 = _seg128(src_p, dst_p, y1).reshape(NC, NP, F)
    y2 = _mid_tc(deg_parts, agg1, y1, b1.reshape(1, F), W2p)
    agg2, _ = _seg64(src_p2, dst_p2, y2)
    agg2 = agg2.reshape(NC, NP, F)
    y3 = _mid_tc(deg_parts, agg2, y2, b2p, W3p)
    agg3, _ = _seg32(src_p2, dst_p2, y3)
    agg3 = agg3.reshape(NC, NP, F)
    pq = _final_tc(deg_parts, agg3, y3, b3p, wc2, bc2)
    scores = _edge_k(src, dst, pq.reshape(2 * NP))
    return scores.reshape(E_IN, 1)


# speedup vs baseline: 30.3824x; 1.0110x over previous
import functools

import jax
import jax.numpy as jnp
import numpy as np
from jax import lax
from jax.experimental import pallas as pl
from jax.experimental.pallas import tpu as pltpu
from jax.experimental.pallas import tpu_sc as plsc

N_IN = 10000
E_IN = 320000
NP = 10240
F = 128
NC, NS, L = 2, 16, 16
NW = NC * NS
C = 64
CPT = 160
HCPT = CPT // 4
PH = CPT // HCPT
EPAD = C * CPT * NW
NB = 2
NIT = HCPT // (2 * NB)
RPT = NP // NS
ZC = 64
EPT = E_IN // NW

_MESH = plsc.VectorSubcoreMesh(core_axis_name="c", subcore_axis_name="s")


def _make_hist():
    @functools.partial(
        pl.kernel,
        out_type=jax.ShapeDtypeStruct((NC * NP,), jnp.float32),
        mesh=_MESH,
        scratch_types=[
            pltpu.VMEM((CPT, C), jnp.int32),
            pltpu.VMEM((C,), jnp.float32),
            pltpu.VMEM((RPT,), jnp.float32),
            pltpu.VMEM_SHARED((NP,), jnp.float32),
        ] + [pltpu.SemaphoreType.DMA] * 8,
    )
    def hist(dst_hbm, deg_hbm, didx, ones, buf, deg_sh, *sems):
        cid = lax.axis_index("c")
        sid = lax.axis_index("s")
        wid = sid * NC + cid

        def zb(t, _):
            buf[pl.ds(t * L, L)] = jnp.zeros((L,), jnp.float32)
            return 0
        lax.fori_loop(0, RPT // L, zb, 0)

        def ob(t, _):
            ones[pl.ds(t * L, L)] = jnp.ones((L,), jnp.float32)
            return 0
        lax.fori_loop(0, C // L, ob, 0)

        pltpu.sync_copy(dst_hbm.at[pl.ds(wid * CPT, CPT)], didx)
        rbase = sid * RPT
        pltpu.sync_copy(buf, deg_sh.at[pl.ds(rbase, RPT)])
        plsc.subcore_barrier()

        def body(g, _):
            for u in range(8):
                pltpu.async_copy(ones, deg_sh.at[didx.at[g * 8 + u]], sems[u], add=True)
            for u in range(8):
                pltpu.make_async_copy(ones, deg_sh.at[didx.at[g * 8 + u]], sems[u]).wait()
            return 0
        lax.fori_loop(0, CPT // 8, body, 0)

        plsc.subcore_barrier()
        pltpu.sync_copy(deg_sh.at[pl.ds(rbase, RPT)], buf)
        pltpu.sync_copy(buf, deg_hbm.at[pl.ds(cid * NP + rbase, RPT)])

    return hist


C2 = 128
CPT2 = EPAD // (C2 * NW)
HCPT2 = CPT2 // 4
NIT2 = HCPT2 // (2 * NB)
OC2 = 32


def _make_segsum_narrow(fr):
    grp = F // L
    ngrp = fr // L

    @functools.partial(
        pl.kernel,
        out_type=[
            jax.ShapeDtypeStruct((NC * NP, F), jnp.float32),
            jax.ShapeDtypeStruct((NP, fr), jnp.float32),
        ],
        mesh=_MESH,
        scratch_types=[
            pltpu.VMEM((HCPT2, C2), jnp.int32),
            pltpu.VMEM((HCPT2, C2), jnp.int32),
            pltpu.VMEM((OC2, F), jnp.float32),
            pltpu.VMEM_SHARED((NP, fr), jnp.float32),
        ] + [pltpu.VMEM((C2, fr), jnp.float32)] * (2 * NB)
          + [pltpu.SemaphoreType.DMA] * (4 * NB),
        compiler_params=pltpu.CompilerParams(use_tc_tiling_on_sc=False),
    )
    def seg(src_hbm, dst_hbm, y_hbm, agg_hbm, yn_hbm, sidx, didx, obuf,
            agg_sh, *rest):
        rows = rest[: 2 * NB]
        semg = rest[2 * NB: 3 * NB]
        semgb = rest[3 * NB: 4 * NB]
        sems = rest[4 * NB: 5 * NB]
        semsb = rest[5 * NB: 6 * NB]
        cid = lax.axis_index("c")
        sid = lax.axis_index("s")
        wid = sid * NC + cid
        rbase = sid * RPT

        def stage(k, _):
            r = rbase + k * OC2
            pltpu.sync_copy(y_hbm.at[pl.ds(r, OC2)], obuf)

            def rep(t, _):
                i = t // ngrp
                j = t % ngrp
                rows[0][i, pl.ds(j * L, L)] = obuf[i, pl.ds(j * L, L)]
                return 0
            lax.fori_loop(0, OC2 * ngrp, rep, 0)
            pltpu.sync_copy(rows[0].at[pl.ds(0, OC2)], yn_hbm.at[pl.ds(r, OC2)])
            return 0
        lax.fori_loop(0, RPT // OC2, stage, 0)

        def zstore(t, _):
            i = t // grp
            j = t % grp
            obuf[i, pl.ds(j * L, L)] = jnp.zeros((L,), jnp.float32)
            return 0
        lax.fori_loop(0, OC2 * grp, zstore, 0)

        def zrow(t, _):
            i = t // ngrp
            j = t % ngrp
            rows[1][i, pl.ds(j * L, L)] = jnp.zeros((L,), jnp.float32)
            return 0
        lax.fori_loop(0, C2 * ngrp, zrow, 0)

        def zcp(k, _):
            pltpu.sync_copy(rows[1], agg_sh.at[pl.ds(rbase + k * C2, C2)])
            return 0
        lax.fori_loop(0, RPT // C2, zcp, 0)
        plsc.subcore_barrier()

        A = rows[:NB]
        B = rows[NB:]

        def g_start(buf, sem, t):
            pltpu.async_copy(yn_hbm.at[sidx.at[t]], buf, sem)

        def g_wait(buf, sem, t):
            pltpu.make_async_copy(yn_hbm.at[sidx.at[t]], buf, sem).wait()

        def s_start(buf, sem, t):
            pltpu.async_copy(buf, agg_sh.at[didx.at[t]], sem, add=True)

        def s_wait(buf, sem, t):
            pltpu.make_async_copy(buf, agg_sh.at[didx.at[t]], sem).wait()

        for ph in range(PH):
            pltpu.sync_copy(src_hbm.at[pl.ds(wid * CPT2 + ph * HCPT2, HCPT2)], sidx)
            pltpu.sync_copy(dst_hbm.at[pl.ds(wid * CPT2 + ph * HCPT2, HCPT2)], didx)
            for b in range(NB):
                g_start(A[b], semg[b], b)

            def body(i, _):
                base = 2 * NB * i
                for b in range(NB):
                    g_wait(A[b], semg[b], base + b)
                    s_start(A[b], sems[b], base + b)

                @pl.when(i > 0)
                def _():
                    for b in range(NB):
                        s_wait(B[b], semsb[b], base - NB + b)
                for b in range(NB):
                    g_start(B[b], semgb[b], base + NB + b)

                for b in range(NB):
                    g_wait(B[b], semgb[b], base + NB + b)
                    s_start(B[b], semsb[b], base + NB + b)

                for b in range(NB):
                    s_wait(A[b], sems[b], base + b)

                @pl.when(i < NIT2 - 1)
                def _():
                    for b in range(NB):
                        g_start(A[b], semg[b], base + 2 * NB + b)
                return 0
            lax.fori_loop(0, NIT2, body, 0)
            for b in range(NB):
                s_wait(B[b], semsb[b], 2 * NB * (NIT2 - 1) + NB + b)

        plsc.subcore_barrier()

        def ocp(k, _):
            r = rbase + k * OC2
            pltpu.sync_copy(agg_sh.at[pl.ds(r, OC2)], rows[0].at[pl.ds(0, OC2)])

            def rep(t, _):
                i = t // ngrp
                j = t % ngrp
                obuf[i, pl.ds(j * L, L)] = rows[0][i, pl.ds(j * L, L)]
                return 0
            lax.fori_loop(0, OC2 * ngrp, rep, 0)
            pltpu.sync_copy(obuf, agg_hbm.at[pl.ds(cid * NP + r, OC2)])
            return 0
        lax.fori_loop(0, RPT // OC2, ocp, 0)

    return seg


def _make_segsum_full():
    grp = F // L

    @functools.partial(
        pl.kernel,
        out_type=jax.ShapeDtypeStruct((NC * NP, F), jnp.float32),
        mesh=_MESH,
        scratch_types=[
            pltpu.VMEM((HCPT, C), jnp.int32),
            pltpu.VMEM((HCPT, C), jnp.int32),
            pltpu.VMEM_SHARED((NP, F), jnp.float32),
        ] + [pltpu.VMEM((C, F), jnp.float32)] * (2 * NB)
          + [pltpu.SemaphoreType.DMA] * (4 * NB),
    )
    def seg(src_hbm, dst_hbm, y_hbm, agg_hbm, sidx, didx, agg_sh, *bufs_sems):
        rows = bufs_sems[: 2 * NB]
        semg = bufs_sems[2 * NB: 3 * NB]
        semgb = bufs_sems[3 * NB: 4 * NB]
        sems = bufs_sems[4 * NB: 5 * NB]
        semsb = bufs_sems[5 * NB: 6 * NB]
        cid = lax.axis_index("c")
        sid = lax.axis_index("s")
        wid = sid * NC + cid

        def zstore(t, _):
            i = t // grp
            j = t % grp
            rows[0][i, pl.ds(j * L, L)] = jnp.zeros((L,), jnp.float32)
            return 0
        lax.fori_loop(0, C * grp, zstore, 0)

        rbase = sid * RPT

        def zcp(k, _):
            pltpu.sync_copy(rows[0], agg_sh.at[pl.ds(rbase + k * ZC, ZC)])
            return 0
        lax.fori_loop(0, RPT // ZC, zcp, 0)
        plsc.subcore_barrier()

        A = rows[:NB]
        B = rows[NB:]

        def g_start(buf, sem, t):
            pltpu.async_copy(y_hbm.at[sidx.at[t]], buf, sem)

        def g_wait(buf, sem, t):
            pltpu.make_async_copy(y_hbm.at[sidx.at[t]], buf, sem).wait()

        def s_start(buf, sem, t):
            pltpu.async_copy(buf, agg_sh.at[didx.at[t]], sem, add=True)

        def s_wait(buf, sem, t):
            pltpu.make_async_copy(buf, agg_sh.at[didx.at[t]], sem).wait()

        for ph in range(PH):
            pltpu.sync_copy(src_hbm.at[pl.ds(wid * CPT + ph * HCPT, HCPT)], sidx)
            pltpu.sync_copy(dst_hbm.at[pl.ds(wid * CPT + ph * HCPT, HCPT)], didx)
            for b in range(NB):
                g_start(A[b], semg[b], b)

            def body(i, _):
                base = 2 * NB * i
                for b in range(NB):
                    g_wait(A[b], semg[b], base + b)
                    s_start(A[b], sems[b], base + b)

                @pl.when(i > 0)
                def _():
                    for b in range(NB):
                        s_wait(B[b], semsb[b], base - NB + b)
                for b in range(NB):
                    g_start(B[b], semgb[b], base + NB + b)

                for b in range(NB):
                    g_wait(B[b], semgb[b], base + NB + b)
                    s_start(B[b], semsb[b], base + NB + b)

                for b in range(NB):
                    s_wait(A[b], sems[b], base + b)

                @pl.when(i < NIT - 1)
                def _():
                    for b in range(NB):
                        g_start(A[b], semg[b], base + 2 * NB + b)
                return 0
            lax.fori_loop(0, NIT, body, 0)
            for b in range(NB):
                s_wait(B[b], semsb[b], 2 * NB * (NIT - 1) + NB + b)

        plsc.subcore_barrier()

        def ocp(k, _):
            r = rbase + k * ZC
            pltpu.sync_copy(agg_sh.at[pl.ds(r, ZC)], rows[0])
            pltpu.sync_copy(rows[0], agg_hbm.at[pl.ds(cid * NP + r, ZC)])
            return 0
        lax.fori_loop(0, RPT // ZC, ocp, 0)

    return seg


def _make_edge():
    @functools.partial(
        pl.kernel,
        out_type=jax.ShapeDtypeStruct((E_IN,), jnp.float32),
        mesh=_MESH,
        scratch_types=[
            pltpu.VMEM((EPT,), jnp.int32),
            pltpu.VMEM((EPT,), jnp.int32),
            pltpu.VMEM((EPT,), jnp.float32),
            pltpu.VMEM((NP,), jnp.float32),
            pltpu.VMEM((NP,), jnp.float32),
        ],
        compiler_params=pltpu.CompilerParams(needs_layout_passes=False),
    )
    def edge(src_hbm, dst_hbm, p_hbm, q_hbm, out_hbm, sidx, didx, outv, pv_v, qv_v):
        cid = lax.axis_index("c")
        sid = lax.axis_index("s")
        wid = sid * NC + cid
        base = wid * EPT
        pltpu.sync_copy(p_hbm, pv_v)
        pltpu.sync_copy(q_hbm, qv_v)
        pltpu.sync_copy(src_hbm.at[pl.ds(base, EPT)], sidx)
        pltpu.sync_copy(dst_hbm.at[pl.ds(base, EPT)], didx)

        def body(i, _):
            s16 = sidx[pl.ds(i * L, L)]
            d16 = didx[pl.ds(i * L, L)]
            pv = plsc.load_gather(pv_v, [s16])
            qv = plsc.load_gather(qv_v, [d16])
            outv[pl.ds(i * L, L)] = pv + qv
            return 0
        lax.fori_loop(0, EPT // L, body, 0)
        pltpu.sync_copy(outv, out_hbm.at[pl.ds(base, EPT)])

    return edge


_R = 1024
_GRID = NP // _R


def _deg_specs():
    return [
        pl.BlockSpec((_R,), lambda i: (i,)),
        pl.BlockSpec((_R,), lambda i: (i + _GRID,)),
    ]


def _dinv_of(deg0_ref, deg1_ref):
    return lax.rsqrt(deg0_ref[...] + deg1_ref[...] + 1.0)


def _prep_tc(deg_flat, x, W1):
    def body(deg0_ref, deg1_ref, x_ref, w_ref, y_ref):
        dinv = _dinv_of(deg0_ref, deg1_ref)
        xw = jnp.dot(x_ref[...], w_ref[...], preferred_element_type=jnp.float32)
        y_ref[...] = xw * dinv[:, None]

    return pl.pallas_call(
        body,
        grid=(_GRID,),
        in_specs=_deg_specs() + [
            pl.BlockSpec((_R, F), lambda i: (i, 0)),
            pl.BlockSpec((F, F), lambda i: (0, 0)),
        ],
        out_specs=pl.BlockSpec((_R, F), lambda i: (i, 0)),
        out_shape=jax.ShapeDtypeStruct((NP, F), jnp.float32),
    )(deg_flat, deg_flat, x, W1)


def _mid_tc(deg_flat, agg, y, b, Wn):
    def body(deg0_ref, deg1_ref, agg_ref, y_ref, b_ref, w_ref, o_ref):
        dinv = _dinv_of(deg0_ref, deg1_ref)
        tot = agg_ref[0] + agg_ref[1] + y_ref[...]
        h = jnp.maximum(tot * dinv[:, None] + b_ref[0, :][None, :], 0.0)
        hw = jnp.dot(h, w_ref[...], preferred_element_type=jnp.float32)
        o_ref[...] = hw * dinv[:, None]

    return pl.pallas_call(
        body,
        grid=(_GRID,),
        in_specs=_deg_specs() + [
            pl.BlockSpec((NC, _R, F), lambda i: (0, i, 0)),
            pl.BlockSpec((_R, F), lambda i: (i, 0)),
            pl.BlockSpec((1, F), lambda i: (0, 0)),
            pl.BlockSpec((F, F), lambda i: (0, 0)),
        ],
        out_specs=pl.BlockSpec((_R, F), lambda i: (i, 0)),
        out_shape=jax.ShapeDtypeStruct((NP, F), jnp.float32),
    )(deg_flat, deg_flat, agg, y, b, Wn)


def _final_tc(deg_flat, agg, y, b, wc2, bc2):
    def body(deg0_ref, deg1_ref, agg_ref, y_ref, b_ref, w_ref, bc_ref,
             p_ref, q_ref):
        dinv = _dinv_of(deg0_ref, deg1_ref)
        tot = agg_ref[0] + agg_ref[1] + y_ref[...]
        h = jnp.maximum(tot * dinv[:, None] + b_ref[0, :][None, :], 0.0)
        pq = jnp.dot(h, w_ref[...], preferred_element_type=jnp.float32)
        pq = pq + bc_ref[0, :][None, :]
        p_ref[...] = pq[:, 0]
        q_ref[...] = pq[:, 1]

    return pl.pallas_call(
        body,
        grid=(_GRID,),
        in_specs=_deg_specs() + [
            pl.BlockSpec((NC, _R, F), lambda i: (0, i, 0)),
            pl.BlockSpec((_R, F), lambda i: (i, 0)),
            pl.BlockSpec((1, F), lambda i: (0, 0)),
            pl.BlockSpec((F, 2), lambda i: (0, 0)),
            pl.BlockSpec((1, 2), lambda i: (0, 0)),
        ],
        out_specs=[
            pl.BlockSpec((_R,), lambda i: (i,)),
            pl.BlockSpec((_R,), lambda i: (i,)),
        ],
        out_shape=[
            jax.ShapeDtypeStruct((NP,), jnp.float32),
            jax.ShapeDtypeStruct((NP,), jnp.float32),
        ],
    )(deg_flat, deg_flat, agg, y, b, wc2, bc2)


_hist_k = _make_hist()
_seg128 = _make_segsum_full()
_seg64 = _make_segsum_narrow(64)
_seg32 = _make_segsum_narrow(32)
_edge_k = _make_edge()


def _pad2(w, rows, cols):
    return jnp.pad(w, ((0, rows - w.shape[0]), (0, cols - w.shape[1])))


_PAD_IDX = (N_IN + (np.arange(EPAD - E_IN) % (NP - N_IN))).astype(np.int32)


def kernel(x, edge_index, W1, b1, W2, b2, W3, b3, Wc, bc):
    src = edge_index[0].astype(jnp.int32)
    dst = edge_index[1].astype(jnp.int32)
    pad_idx = jnp.asarray(_PAD_IDX)
    src_p = jnp.concatenate([src, pad_idx]).reshape(NW * CPT, C)
    dst_p = jnp.concatenate([dst, pad_idx]).reshape(NW * CPT, C)
    W2p = _pad2(W2, F, F)
    W3p = _pad2(W3, F, F)
    b2p = jnp.pad(b2, (0, F - b2.shape[0])).reshape(1, F)
    b3p = jnp.pad(b3, (0, F - b3.shape[0])).reshape(1, F)
    wc2 = _pad2(jnp.concatenate([Wc[:32], Wc[32:]], axis=1), F, 2)
    bc2 = jnp.concatenate([bc, jnp.zeros((1,), jnp.float32)]).reshape(1, 2)

    deg_flat = _hist_k(dst_p)
    y1 = _prep_tc(deg_flat, x, W1)
    src_p2 = src_p.reshape(NW * CPT2, C2)
    dst_p2 = dst_p.reshape(NW * CPT2, C2)
    agg1 = _seg128(src_p, dst_p, y1).reshape(NC, NP, F)
    y2 = _mid_tc(deg_flat, agg1, y1, b1.reshape(1, F), W2p)
    agg2, _ = _seg64(src_p2, dst_p2, y2)
    agg2 = agg2.reshape(NC, NP, F)
    y3 = _mid_tc(deg_flat, agg2, y2, b2p, W3p)
    agg3, _ = _seg32(src_p2, dst_p2, y3)
    agg3 = agg3.reshape(NC, NP, F)
    p, q = _final_tc(deg_flat, agg3, y3, b3p, wc2, bc2)
    scores = _edge_k(src, dst, p, q)
    return scores.reshape(E_IN, 1)


# speedup vs baseline: 30.3892x; 1.0002x over previous
import functools

import jax
import jax.numpy as jnp
import numpy as np
from jax import lax
from jax.experimental import pallas as pl
from jax.experimental.pallas import tpu as pltpu
from jax.experimental.pallas import tpu_sc as plsc

N_IN = 10000
E_IN = 320000
NP = 10240
F = 128
NC, NS, L = 2, 16, 16
NW = NC * NS
C = 64
CPT = 160
HCPT = CPT // 4
PH = CPT // HCPT
EPAD = C * CPT * NW
NB = 2
NIT = HCPT // (2 * NB)
RPT = NP // NS
ZC = 64
EPT = E_IN // NW

_MESH = plsc.VectorSubcoreMesh(
    core_axis_name="c", subcore_axis_name="s", num_cores=NC, num_subcores=NS)


def _make_hist():
    @functools.partial(
        pl.kernel,
        out_type=jax.ShapeDtypeStruct((NC * NP,), jnp.float32),
        mesh=_MESH,
        scratch_types=[
            pltpu.VMEM((CPT, C), jnp.int32),
            pltpu.VMEM((C,), jnp.float32),
            pltpu.VMEM((RPT,), jnp.float32),
            pltpu.VMEM_SHARED((NP,), jnp.float32),
        ] + [pltpu.SemaphoreType.DMA] * 8,
    )
    def hist(dst_hbm, deg_hbm, didx, ones, buf, deg_sh, *sems):
        cid = lax.axis_index("c")
        sid = lax.axis_index("s")
        wid = sid * NC + cid

        def zb(t, _):
            buf[pl.ds(t * L, L)] = jnp.zeros((L,), jnp.float32)
            return 0
        lax.fori_loop(0, RPT // L, zb, 0)

        def ob(t, _):
            ones[pl.ds(t * L, L)] = jnp.ones((L,), jnp.float32)
            return 0
        lax.fori_loop(0, C // L, ob, 0)

        pltpu.sync_copy(dst_hbm.at[pl.ds(wid * CPT, CPT)], didx)
        rbase = sid * RPT
        pltpu.sync_copy(buf, deg_sh.at[pl.ds(rbase, RPT)])
        plsc.subcore_barrier()

        def body(g, _):
            for u in range(8):
                pltpu.async_copy(ones, deg_sh.at[didx.at[g * 8 + u]], sems[u], add=True)
            for u in range(8):
                pltpu.make_async_copy(ones, deg_sh.at[didx.at[g * 8 + u]], sems[u]).wait()
            return 0
        lax.fori_loop(0, CPT // 8, body, 0)

        plsc.subcore_barrier()
        pltpu.sync_copy(deg_sh.at[pl.ds(rbase, RPT)], buf)
        pltpu.sync_copy(buf, deg_hbm.at[pl.ds(cid * NP + rbase, RPT)])

    return hist


C2 = 128
CPT2 = EPAD // (C2 * NW)
HCPT2 = CPT2 // 4
NIT2 = HCPT2 // (2 * NB)
OC2 = 32


def _make_segsum_narrow(fr):
    grp = F // L
    ngrp = fr // L

    @functools.partial(
        pl.kernel,
        out_type=[
            jax.ShapeDtypeStruct((NC * NP, F), jnp.float32),
            jax.ShapeDtypeStruct((NP, fr), jnp.float32),
        ],
        mesh=_MESH,
        scratch_types=[
            pltpu.VMEM((HCPT2, C2), jnp.int32),
            pltpu.VMEM((HCPT2, C2), jnp.int32),
            pltpu.VMEM((OC2, F), jnp.float32),
            pltpu.VMEM_SHARED((NP, fr), jnp.float32),
        ] + [pltpu.VMEM((C2, fr), jnp.float32)] * (2 * NB)
          + [pltpu.SemaphoreType.DMA] * (4 * NB),
        compiler_params=pltpu.CompilerParams(use_tc_tiling_on_sc=False),
    )
    def seg(src_hbm, dst_hbm, y_hbm, agg_hbm, yn_hbm, sidx, didx, obuf,
            agg_sh, *rest):
        rows = rest[: 2 * NB]
        semg = rest[2 * NB: 3 * NB]
        semgb = rest[3 * NB: 4 * NB]
        sems = rest[4 * NB: 5 * NB]
        semsb = rest[5 * NB: 6 * NB]
        cid = lax.axis_index("c")
        sid = lax.axis_index("s")
        wid = sid * NC + cid
        rbase = sid * RPT

        def stage(k, _):
            r = rbase + k * OC2
            pltpu.sync_copy(y_hbm.at[pl.ds(r, OC2)], obuf)

            def rep(t, _):
                i = t // ngrp
                j = t % ngrp
                rows[0][i, pl.ds(j * L, L)] = obuf[i, pl.ds(j * L, L)]
                return 0
            lax.fori_loop(0, OC2 * ngrp, rep, 0)
            pltpu.sync_copy(rows[0].at[pl.ds(0, OC2)], yn_hbm.at[pl.ds(r, OC2)])
            return 0
        lax.fori_loop(0, RPT // OC2, stage, 0)

        def zstore(t, _):
            i = t // grp
            j = t % grp
            obuf[i, pl.ds(j * L, L)] = jnp.zeros((L,), jnp.float32)
            return 0
        lax.fori_loop(0, OC2 * grp, zstore, 0)

        def zrow(t, _):
            i = t // ngrp
            j = t % ngrp
            rows[1][i, pl.ds(j * L, L)] = jnp.zeros((L,), jnp.float32)
            return 0
        lax.fori_loop(0, C2 * ngrp, zrow, 0)

        def zcp(k, _):
            pltpu.sync_copy(rows[1], agg_sh.at[pl.ds(rbase + k * C2, C2)])
            return 0
        lax.fori_loop(0, RPT // C2, zcp, 0)
        plsc.subcore_barrier()

        A = rows[:NB]
        B = rows[NB:]

        def g_start(buf, sem, t):
            pltpu.async_copy(yn_hbm.at[sidx.at[t]], buf, sem)

        def g_wait(buf, sem, t):
            pltpu.make_async_copy(yn_hbm.at[sidx.at[t]], buf, sem).wait()

        def s_start(buf, sem, t):
            pltpu.async_copy(buf, agg_sh.at[didx.at[t]], sem, add=True)

        def s_wait(buf, sem, t):
            pltpu.make_async_copy(buf, agg_sh.at[didx.at[t]], sem).wait()

        for ph in range(PH):
            pltpu.sync_copy(src_hbm.at[pl.ds(wid * CPT2 + ph * HCPT2, HCPT2)], sidx)
            pltpu.sync_copy(dst_hbm.at[pl.ds(wid * CPT2 + ph * HCPT2, HCPT2)], didx)
            for b in range(NB):
                g_start(A[b], semg[b], b)

            def body(i, _):
                base = 2 * NB * i
                for b in range(NB):
                    g_wait(A[b], semg[b], base + b)
                    s_start(A[b], sems[b], base + b)

                @pl.when(i > 0)
                def _():
                    for b in range(NB):
                        s_wait(B[b], semsb[b], base - NB + b)
                for b in range(NB):
                    g_start(B[b], semgb[b], base + NB + b)

                for b in range(NB):
                    g_wait(B[b], semgb[b], base + NB + b)
                    s_start(B[b], semsb[b], base + NB + b)

                for b in range(NB):
                    s_wait(A[b], sems[b], base + b)

                @pl.when(i < NIT2 - 1)
                def _():
                    for b in range(NB):
                        g_start(A[b], semg[b], base + 2 * NB + b)
                return 0
            lax.fori_loop(0, NIT2, body, 0)
            for b in range(NB):
                s_wait(B[b], semsb[b], 2 * NB * (NIT2 - 1) + NB + b)

        plsc.subcore_barrier()

        def ocp(k, _):
            r = rbase + k * OC2
            pltpu.sync_copy(agg_sh.at[pl.ds(r, OC2)], rows[0].at[pl.ds(0, OC2)])

            def rep(t, _):
                i = t // ngrp
                j = t % ngrp
                obuf[i, pl.ds(j * L, L)] = rows[0][i, pl.ds(j * L, L)]
                return 0
            lax.fori_loop(0, OC2 * ngrp, rep, 0)
            pltpu.sync_copy(obuf, agg_hbm.at[pl.ds(cid * NP + r, OC2)])
            return 0
        lax.fori_loop(0, RPT // OC2, ocp, 0)

    return seg


def _make_segsum_full():
    grp = F // L

    @functools.partial(
        pl.kernel,
        out_type=jax.ShapeDtypeStruct((NC * NP, F), jnp.float32),
        mesh=_MESH,
        scratch_types=[
            pltpu.VMEM((HCPT, C), jnp.int32),
            pltpu.VMEM((HCPT, C), jnp.int32),
            pltpu.VMEM_SHARED((NP, F), jnp.float32),
        ] + [pltpu.VMEM((C, F), jnp.float32)] * (2 * NB)
          + [pltpu.SemaphoreType.DMA] * (4 * NB),
    )
    def seg(src_hbm, dst_hbm, y_hbm, agg_hbm, sidx, didx, agg_sh, *bufs_sems):
        rows = bufs_sems[: 2 * NB]
        semg = bufs_sems[2 * NB: 3 * NB]
        semgb = bufs_sems[3 * NB: 4 * NB]
        sems = bufs_sems[4 * NB: 5 * NB]
        semsb = bufs_sems[5 * NB: 6 * NB]
        cid = lax.axis_index("c")
        sid = lax.axis_index("s")
        wid = sid * NC + cid

        def zstore(t, _):
            i = t // grp
            j = t % grp
            rows[0][i, pl.ds(j * L, L)] = jnp.zeros((L,), jnp.float32)
            return 0
        lax.fori_loop(0, C * grp, zstore, 0)

        rbase = sid * RPT

        def zcp(k, _):
            pltpu.sync_copy(rows[0], agg_sh.at[pl.ds(rbase + k * ZC, ZC)])
            return 0
        lax.fori_loop(0, RPT // ZC, zcp, 0)
        plsc.subcore_barrier()

        A = rows[:NB]
        B = rows[NB:]

        def g_start(buf, sem, t):
            pltpu.async_copy(y_hbm.at[sidx.at[t]], buf, sem)

        def g_wait(buf, sem, t):
            pltpu.make_async_copy(y_hbm.at[sidx.at[t]], buf, sem).wait()

        def s_start(buf, sem, t):
            pltpu.async_copy(buf, agg_sh.at[didx.at[t]], sem, add=True)

        def s_wait(buf, sem, t):
            pltpu.make_async_copy(buf, agg_sh.at[didx.at[t]], sem).wait()

        for ph in range(PH):
            pltpu.sync_copy(src_hbm.at[pl.ds(wid * CPT + ph * HCPT, HCPT)], sidx)
            pltpu.sync_copy(dst_hbm.at[pl.ds(wid * CPT + ph * HCPT, HCPT)], didx)
            for b in range(NB):
                g_start(A[b], semg[b], b)

            def body(i, _):
                base = 2 * NB * i
                for b in range(NB):
                    g_wait(A[b], semg[b], base + b)
                    s_start(A[b], sems[b], base + b)

                @pl.when(i > 0)
                def _():
                    for b in range(NB):
                        s_wait(B[b], semsb[b], base - NB + b)
                for b in range(NB):
                    g_start(B[b], semgb[b], base + NB + b)

                for b in range(NB):
                    g_wait(B[b], semgb[b], base + NB + b)
                    s_start(B[b], semsb[b], base + NB + b)

                for b in range(NB):
                    s_wait(A[b], sems[b], base + b)

                @pl.when(i < NIT - 1)
                def _():
                    for b in range(NB):
                        g_start(A[b], semg[b], base + 2 * NB + b)
                return 0
            lax.fori_loop(0, NIT, body, 0)
            for b in range(NB):
                s_wait(B[b], semsb[b], 2 * NB * (NIT - 1) + NB + b)

        plsc.subcore_barrier()

        def ocp(k, _):
            r = rbase + k * ZC
            pltpu.sync_copy(agg_sh.at[pl.ds(r, ZC)], rows[0])
            pltpu.sync_copy(rows[0], agg_hbm.at[pl.ds(cid * NP + r, ZC)])
            return 0
        lax.fori_loop(0, RPT // ZC, ocp, 0)

    return seg


def _make_edge():
    @functools.partial(
        pl.kernel,
        out_type=jax.ShapeDtypeStruct((E_IN,), jnp.float32),
        mesh=_MESH,
        scratch_types=[
            pltpu.VMEM((EPT,), jnp.int32),
            pltpu.VMEM((EPT,), jnp.int32),
            pltpu.VMEM((EPT,), jnp.float32),
            pltpu.VMEM((NP,), jnp.float32),
            pltpu.VMEM((NP,), jnp.float32),
        ],
        compiler_params=pltpu.CompilerParams(needs_layout_passes=False),
    )
    def edge(src_hbm, dst_hbm, p_hbm, q_hbm, out_hbm, sidx, didx, outv, pv_v, qv_v):
        cid = lax.axis_index("c")
        sid = lax.axis_index("s")
        wid = sid * NC + cid
        base = wid * EPT
        pltpu.sync_copy(p_hbm, pv_v)
        pltpu.sync_copy(q_hbm, qv_v)
        pltpu.sync_copy(src_hbm.at[pl.ds(base, EPT)], sidx)
        pltpu.sync_copy(dst_hbm.at[pl.ds(base, EPT)], didx)

        def body(i, _):
            s16 = sidx[pl.ds(i * L, L)]
            d16 = didx[pl.ds(i * L, L)]
            pv = plsc.load_gather(pv_v, [s16])
            qv = plsc.load_gather(qv_v, [d16])
            outv[pl.ds(i * L, L)] = pv + qv
            return 0
        lax.fori_loop(0, EPT // L, body, 0)
        pltpu.sync_copy(outv, out_hbm.at[pl.ds(base, EPT)])

    return edge


_R = 1024
_GRID = NP // _R


def _deg_specs():
    return [
        pl.BlockSpec((_R,), lambda i: (i,)),
        pl.BlockSpec((_R,), lambda i: (i + _GRID,)),
    ]


def _dinv_of(deg0_ref, deg1_ref):
    return lax.rsqrt(deg0_ref[...] + deg1_ref[...] + 1.0)


def _prep_tc(deg_flat, x, W1):
    def body(deg0_ref, deg1_ref, x_ref, w_ref, y_ref):
        dinv = _dinv_of(deg0_ref, deg1_ref)
        xw = jnp.dot(x_ref[...], w_ref[...], preferred_element_type=jnp.float32)
        y_ref[...] = xw * dinv[:, None]

    return pl.pallas_call(
        body,
        grid=(_GRID,),
        in_specs=_deg_specs() + [
            pl.BlockSpec((_R, F), lambda i: (i, 0)),
            pl.BlockSpec((F, F), lambda i: (0, 0)),
        ],
        out_specs=pl.BlockSpec((_R, F), lambda i: (i, 0)),
        out_shape=jax.ShapeDtypeStruct((NP, F), jnp.float32),
    )(deg_flat, deg_flat, x, W1)


def _mid_tc(deg_flat, agg, y, b, Wn):
    def body(deg0_ref, deg1_ref, agg_ref, y_ref, b_ref, w_ref, o_ref):
        dinv = _dinv_of(deg0_ref, deg1_ref)
        tot = agg_ref[0] + agg_ref[1] + y_ref[...]
        h = jnp.maximum(tot * dinv[:, None] + b_ref[0, :][None, :], 0.0)
        hw = jnp.dot(h, w_ref[...], preferred_element_type=jnp.float32)
        o_ref[...] = hw * dinv[:, None]

    return pl.pallas_call(
        body,
        grid=(_GRID,),
        in_specs=_deg_specs() + [
            pl.BlockSpec((NC, _R, F), lambda i: (0, i, 0)),
            pl.BlockSpec((_R, F), lambda i: (i, 0)),
            pl.BlockSpec((1, F), lambda i: (0, 0)),
            pl.BlockSpec((F, F), lambda i: (0, 0)),
        ],
        out_specs=pl.BlockSpec((_R, F), lambda i: (i, 0)),
        out_shape=jax.ShapeDtypeStruct((NP, F), jnp.float32),
    )(deg_flat, deg_flat, agg, y, b, Wn)


def _final_tc(deg_flat, agg, y, b, wc2, bc2):
    def body(deg0_ref, deg1_ref, agg_ref, y_ref, b_ref, w_ref, bc_ref,
             p_ref, q_ref):
        dinv = _dinv_of(deg0_ref, deg1_ref)
        tot = agg_ref[0] + agg_ref[1] + y_ref[...]
        h = jnp.maximum(tot * dinv[:, None] + b_ref[0, :][None, :], 0.0)
        pq = jnp.dot(h, w_ref[...], preferred_element_type=jnp.float32)
        pq = pq + bc_ref[0, :][None, :]
        p_ref[...] = pq[:, 0]
        q_ref[...] = pq[:, 1]

    return pl.pallas_call(
        body,
        grid=(_GRID,),
        in_specs=_deg_specs() + [
            pl.BlockSpec((NC, _R, F), lambda i: (0, i, 0)),
            pl.BlockSpec((_R, F), lambda i: (i, 0)),
            pl.BlockSpec((1, F), lambda i: (0, 0)),
            pl.BlockSpec((F, 2), lambda i: (0, 0)),
            pl.BlockSpec((1, 2), lambda i: (0, 0)),
        ],
        out_specs=[
            pl.BlockSpec((_R,), lambda i: (i,)),
            pl.BlockSpec((_R,), lambda i: (i,)),
        ],
        out_shape=[
            jax.ShapeDtypeStruct((NP,), jnp.float32),
            jax.ShapeDtypeStruct((NP,), jnp.float32),
        ],
    )(deg_flat, deg_flat, agg, y, b, wc2, bc2)


_hist_k = _make_hist()
_seg128 = _make_segsum_full()
_seg64 = _make_segsum_narrow(64)
_seg32 = _make_segsum_narrow(32)
_edge_k = _make_edge()


def _pad2(w, rows, cols):
    return jnp.pad(w, ((0, rows - w.shape[0]), (0, cols - w.shape[1])))


_PAD_IDX = (N_IN + (np.arange(EPAD - E_IN) % (NP - N_IN))).astype(np.int32)


def kernel(x, edge_index, W1, b1, W2, b2, W3, b3, Wc, bc):
    src = edge_index[0].astype(jnp.int32)
    dst = edge_index[1].astype(jnp.int32)
    pad_idx = jnp.asarray(_PAD_IDX)
    src_p = jnp.concatenate([src, pad_idx]).reshape(NW * CPT, C)
    dst_p = jnp.concatenate([dst, pad_idx]).reshape(NW * CPT, C)
    W2p = _pad2(W2, F, F)
    W3p = _pad2(W3, F, F)
    b2p = jnp.pad(b2, (0, F - b2.shape[0])).reshape(1, F)
    b3p = jnp.pad(b3, (0, F - b3.shape[0])).reshape(1, F)
    wc2 = _pad2(jnp.concatenate([Wc[:32], Wc[32:]], axis=1), F, 2)
    bc2 = jnp.concatenate([bc, jnp.zeros((1,), jnp.float32)]).reshape(1, 2)

    deg_flat = _hist_k(dst_p)
    y1 = _prep_tc(deg_flat, x, W1)
    src_p2 = src_p.reshape(NW * CPT2, C2)
    dst_p2 = dst_p.reshape(NW * CPT2, C2)
    agg1 = _seg128(src_p, dst_p, y1).reshape(NC, NP, F)
    y2 = _mid_tc(deg_flat, agg1, y1, b1.reshape(1, F), W2p)
    agg2, _ = _seg64(src_p2, dst_p2, y2)
    agg2 = agg2.reshape(NC, NP, F)
    y3 = _mid_tc(deg_flat, agg2, y2, b2p, W3p)
    agg3, _ = _seg32(src_p2, dst_p2, y3)
    agg3 = agg3.reshape(NC, NP, F)
    p, q = _final_tc(deg_flat, agg3, y3, b3p, wc2, bc2)
    scores = _edge_k(src, dst, p, q)
    return scores.reshape(E_IN, 1)
